# Initial kernel scaffold; baseline (speedup 1.0000x reference)
import functools

import jax
import jax.numpy as jnp
from jax import lax
from jax.experimental import pallas as pl
from jax.experimental.pallas import tpu as pltpu
from jax.experimental.pallas import tpu_sc as plsc

_P = 2147483647
_M = 100000
_A = (48271, 16807)
_B = (12345, 67890)
_D = 64
_NC = 2
_NS = 16
_NW = _NC * _NS
_L = 16
_CHUNK = 128


def _hashes(xv):
    xl = xv & 0x7FFF
    xh = lax.shift_right_logical(xv, 15)
    out = []
    for a, b in zip(_A, _B):
        u = xh * a
        t = (lax.shift_right_logical(u, 16)
             + lax.shift_left(u & 0xFFFF, 15)
             + xl * a + b)
        r = (t & 0x7FFFFFFF) + lax.shift_right_logical(t, 31)
        r = jnp.where(r == _P, 0, jnp.where(r < 0, 1, r))
        h = lax.rem(r, _M) + 4
        out.append(jnp.where(xv < 4, xv, h))
    return out


def _make_lookup(n_tokens):
    t_per_w = n_tokens // _NW
    n_chunks = t_per_w // _CHUNK
    assert t_per_w * _NW == n_tokens and n_chunks * _CHUNK == t_per_w

    mesh = plsc.VectorSubcoreMesh(core_axis_name="c", subcore_axis_name="s")

    @functools.partial(
        pl.kernel,
        out_type=jax.ShapeDtypeStruct((n_tokens, _D), jnp.float32),
        mesh=mesh,
        scratch_types=[
            pltpu.VMEM((_CHUNK,), jnp.int32),
            pltpu.VMEM((_CHUNK,), jnp.int32),
            pltpu.VMEM((_CHUNK,), jnp.int32),
            pltpu.VMEM((_CHUNK, 2), jnp.float32),
            pltpu.VMEM((_CHUNK, _D), jnp.float32),
            pltpu.VMEM((_CHUNK, _D), jnp.float32),
            pltpu.VMEM((_CHUNK, _D), jnp.float32),
            pltpu.SemaphoreType.DMA,
            pltpu.SemaphoreType.DMA,
            pltpu.SemaphoreType.DMA,
        ],
    )
    def lookup(x_hbm, emb_hbm, hw_hbm, out_hbm,
               idx_v, h0_v, h1_v, w_v, r0_v, r1_v, o_v,
               sem_w, sem0, sem1):
        wid = lax.axis_index("s") * _NC + lax.axis_index("c")

        def chunk_body(g, carry):
            base = wid * t_per_w + g * _CHUNK
            pltpu.sync_copy(x_hbm.at[pl.ds(base, _CHUNK)], idx_v)
            cp_w = pltpu.async_copy(hw_hbm.at[idx_v], w_v, sem_w)
            for i in range(_CHUNK // _L):
                sl = pl.ds(i * _L, _L)
                h0, h1 = _hashes(idx_v[sl])
                h0_v[sl] = h0
                h1_v[sl] = h1
            cp0 = pltpu.async_copy(emb_hbm.at[h0_v], r0_v, sem0)
            cp1 = pltpu.async_copy(emb_hbm.at[h1_v], r1_v, sem1)
            cp_w.wait()
            cp0.wait()
            cp1.wait()

            def tok_body(c, carry2):
                w0 = w_v[c, 0]
                w1 = w_v[c, 1]
                for d in range(_D // _L):
                    ds = pl.ds(d * _L, _L)
                    o_v[c, ds] = w0 * r0_v[c, ds] + w1 * r1_v[c, ds]
                return carry2

            lax.fori_loop(0, _CHUNK, tok_body, 0)
            pltpu.sync_copy(o_v, out_hbm.at[pl.ds(base, _CHUNK)])
            return carry

        lax.fori_loop(0, n_chunks, chunk_body, 0)

    return lookup


def kernel(x, shared_embedding, hash_weights):
    b, s = x.shape
    n_tokens = b * s
    xf = x.reshape(n_tokens).astype(jnp.int32)
    lookup = _make_lookup(n_tokens)
    out = lookup(xf, shared_embedding, hash_weights)
    return out.reshape(b, s, _D)


# speedup vs baseline: 1.3752x; 1.3752x over previous
import functools

import jax
import jax.numpy as jnp
from jax import lax
from jax.experimental import pallas as pl
from jax.experimental.pallas import tpu as pltpu
from jax.experimental.pallas import tpu_sc as plsc

_P = 2147483647
_M = 100000
_A = (48271, 16807)
_B = (12345, 67890)
_D = 64
_NC = 2
_NS = 16
_NW = _NC * _NS
_L = 16
_CHUNK = 128


def _hashes(xv):
    i32 = jnp.int32
    xl = xv & i32(0x7FFF)
    xh = lax.shift_right_logical(xv, i32(15))
    out = []
    for a, b in zip(_A, _B):
        u = xh * i32(a)
        t = (lax.shift_right_logical(u, i32(16))
             + lax.shift_left(u & i32(0xFFFF), i32(15))
             + xl * i32(a) + i32(b))
        r = (t & i32(0x7FFFFFFF)) + lax.shift_right_logical(t, i32(31))
        r = jnp.where(r == i32(_P), i32(0), jnp.where(r < i32(0), i32(1), r))
        h = lax.rem(r, i32(_M)) + i32(4)
        out.append(jnp.where(xv < i32(4), xv, h))
    return out


def _make_lookup(n_tokens, chunk=_CHUNK, interpret=False):
    _CHUNK = chunk
    t_per_w = n_tokens // _NW
    n_chunks = t_per_w // _CHUNK
    assert t_per_w * _NW == n_tokens and n_chunks * _CHUNK == t_per_w

    mesh = plsc.VectorSubcoreMesh(
        core_axis_name="c", subcore_axis_name="s",
        num_cores=_NC, num_subcores=_NS)

    @functools.partial(
        pl.kernel,
        out_type=jax.ShapeDtypeStruct((n_tokens, _D), jnp.float32),
        mesh=mesh,
        scratch_types=[
            pltpu.VMEM((_CHUNK,), jnp.int32),
            pltpu.VMEM((_CHUNK,), jnp.int32),
            pltpu.VMEM((_CHUNK,), jnp.int32),
            pltpu.VMEM((_CHUNK,), jnp.int32),
            pltpu.VMEM((_CHUNK, _L), jnp.float32),
            pltpu.VMEM((_CHUNK,), jnp.float32),
            pltpu.VMEM((_CHUNK,), jnp.float32),
            pltpu.VMEM((_CHUNK, _D), jnp.float32),
            pltpu.VMEM((_CHUNK, _D), jnp.float32),
            pltpu.VMEM((_CHUNK, _D), jnp.float32),
            pltpu.SemaphoreType.DMA,
            pltpu.SemaphoreType.DMA,
            pltpu.SemaphoreType.DMA,
        ],
        compiler_params=pltpu.CompilerParams(
            needs_layout_passes=False, use_tc_tiling_on_sc=False),
        interpret=interpret,
    )
    def lookup(x_hbm, emb_hbm, hw_hbm, out_hbm,
               idx_v, h0_v, h1_v, g_v, wraw_v, w0_v, w1_v,
               r0_v, r1_v, o_v, sem_w, sem0, sem1):
        wid = lax.axis_index("s") * _NC + lax.axis_index("c")

        def chunk_body(g, base):
            base = pl.multiple_of(base, _CHUNK)
            pltpu.sync_copy(x_hbm.at[pl.ds(base, _CHUNK)], idx_v)
            for i in range(_CHUNK // _L):
                sl = pl.ds(i * _L, _L)
                xv = idx_v[sl]
                h0, h1 = _hashes(xv)
                h0_v[sl] = h0
                h1_v[sl] = h1
                g_v[sl] = lax.shift_right_logical(xv, jnp.int32(3))
            cp_w = pltpu.async_copy(hw_hbm.at[g_v], wraw_v, sem_w)
            cp0 = pltpu.async_copy(emb_hbm.at[h0_v], r0_v, sem0)
            cp1 = pltpu.async_copy(emb_hbm.at[h1_v], r1_v, sem1)
            cp_w.wait()
            iota = lax.iota(jnp.int32, _L)
            for i in range(_CHUNK // _L):
                sl = pl.ds(i * _L, _L)
                tokv = iota + jnp.int32(i * _L)
                col = lax.shift_left(idx_v[sl] & jnp.int32(7), jnp.int32(1))
                w0_v[sl] = plsc.load_gather(wraw_v, [tokv, col])
                w1_v[sl] = plsc.load_gather(wraw_v, [tokv, col + jnp.int32(1)])
            cp0.wait()
            cp1.wait()

            def tok_body(g2, c):
                csplat = jnp.full((_L,), c, jnp.int32)
                w0 = plsc.load_gather(w0_v, [csplat])
                w1 = plsc.load_gather(w1_v, [csplat])
                for d in range(_D // _L):
                    ds = pl.ds(d * _L, _L)
                    o_v[c, ds] = w0 * r0_v[c, ds] + w1 * r1_v[c, ds]
                return c + jnp.int32(1)

            lax.fori_loop(0, _CHUNK, tok_body, jnp.int32(0))
            pltpu.sync_copy(o_v, out_hbm.at[pl.ds(base, _CHUNK)])
            return base + jnp.int32(_CHUNK)

        lax.fori_loop(0, n_chunks, chunk_body, wid * jnp.int32(t_per_w))

    return lookup


def kernel(x, shared_embedding, hash_weights):
    b, s = x.shape
    n_tokens = b * s
    xf = x.reshape(n_tokens).astype(jnp.int32)
    hw16 = hash_weights.reshape(-1, _L)
    lookup = _make_lookup(n_tokens)
    out = lookup(xf, shared_embedding, hw16)
    return out.reshape(b, s, _D)


# speedup vs baseline: 1.4302x; 1.0400x over previous
import functools

import jax
import jax.numpy as jnp
from jax import lax
from jax.experimental import pallas as pl
from jax.experimental.pallas import tpu as pltpu
from jax.experimental.pallas import tpu_sc as plsc

_P = 2147483647
_M = 100000
_A = (48271, 16807)
_B = (12345, 67890)
_D = 64
_NC = 2
_NS = 16
_NW = _NC * _NS
_L = 16
_CHUNK = 128


def _hashes(xv):
    i32 = jnp.int32
    xl = xv & i32(0x7FFF)
    xh = lax.shift_right_logical(xv, i32(15))
    out = []
    for a, b in zip(_A, _B):
        u = xh * i32(a)
        t = (lax.shift_right_logical(u, i32(16))
             + lax.shift_left(u & i32(0xFFFF), i32(15))
             + xl * i32(a) + i32(b))
        r = (t & i32(0x7FFFFFFF)) + lax.shift_right_logical(t, i32(31))
        r = jnp.where(r == i32(_P), i32(0), jnp.where(r < i32(0), i32(1), r))
        h = lax.rem(r, i32(_M)) + i32(4)
        out.append(jnp.where(xv < i32(4), xv, h))
    return out


def _make_lookup(n_tokens):
    t_per_w = n_tokens // _NW
    n_chunks = t_per_w // _CHUNK
    assert t_per_w * _NW == n_tokens and n_chunks * _CHUNK == t_per_w
    assert n_chunks >= 2 and n_chunks % 2 == 0

    mesh = plsc.VectorSubcoreMesh(
        core_axis_name="c", subcore_axis_name="s",
        num_cores=_NC, num_subcores=_NS)

    @functools.partial(
        pl.kernel,
        out_type=jax.ShapeDtypeStruct((n_tokens, _D), jnp.float32),
        mesh=mesh,
        scratch_types=[
            pltpu.VMEM((t_per_w,), jnp.int32),
            pltpu.VMEM((t_per_w,), jnp.int32),
            pltpu.VMEM((t_per_w,), jnp.int32),
            pltpu.VMEM((t_per_w,), jnp.int32),
            pltpu.VMEM((2, _CHUNK, _L), jnp.float32),
            pltpu.VMEM((2, _CHUNK), jnp.float32),
            pltpu.VMEM((2, _CHUNK), jnp.float32),
            pltpu.VMEM((2, _CHUNK, _D), jnp.float32),
            pltpu.VMEM((2, _CHUNK, _D), jnp.float32),
            pltpu.VMEM((2, _CHUNK, _D), jnp.float32),
            pltpu.SemaphoreType.DMA,
            pltpu.SemaphoreType.DMA,
            pltpu.SemaphoreType.DMA,
            pltpu.SemaphoreType.DMA,
            pltpu.SemaphoreType.DMA,
            pltpu.SemaphoreType.DMA,
            pltpu.SemaphoreType.DMA,
            pltpu.SemaphoreType.DMA,
        ],
        compiler_params=pltpu.CompilerParams(
            needs_layout_passes=False, use_tc_tiling_on_sc=False),
    )
    def lookup(x_hbm, emb_hbm, hw_hbm, out_hbm,
               idx_all, h0_all, h1_all, g_all, wraw_v, w0_v, w1_v,
               r0_v, r1_v, o_v,
               sw0, sw1, s00, s01, s10, s11, so0, so1):
        i32 = jnp.int32
        wid = lax.axis_index("s") * i32(_NC) + lax.axis_index("c")
        tbase = pl.multiple_of(wid * i32(t_per_w), _CHUNK)
        sems = ((sw0, s00, s10, so0), (sw1, s01, s11, so1))

        pltpu.sync_copy(x_hbm.at[pl.ds(tbase, t_per_w)], idx_all)

        groups_per_iter = 16

        def hash_body(_, off):
            for k in range(groups_per_iter):
                sl = pl.ds(off + i32(k * _L), _L)
                xv = idx_all[sl]
                h0, h1 = _hashes(xv)
                h0_all[sl] = h0
                h1_all[sl] = h1
                g_all[sl] = lax.shift_right_logical(xv, i32(3))
            return off + i32(groups_per_iter * _L)

        lax.fori_loop(0, t_per_w // (_L * groups_per_iter), hash_body, i32(0))

        def gather_args(off, b):
            lsl = pl.ds(pl.multiple_of(off, _CHUNK), _CHUNK)
            return ((hw_hbm.at[g_all.at[lsl]], wraw_v.at[i32(b)], sems[b][0]),
                    (emb_hbm.at[h0_all.at[lsl]], r0_v.at[i32(b)], sems[b][1]),
                    (emb_hbm.at[h1_all.at[lsl]], r1_v.at[i32(b)], sems[b][2]))

        def issue_gathers(off, b):
            for args in gather_args(off, b):
                pltpu.async_copy(*args)

        def wait_gathers(off, b):
            for args in gather_args(off, b):
                pltpu.make_async_copy(*args).wait()

        def out_args(off, b):
            osl = pl.ds(pl.multiple_of(tbase + off, _CHUNK), _CHUNK)
            return (o_v.at[i32(b)], out_hbm.at[osl], sems[b][3])

        def combine(off, b):
            iota = lax.iota(i32, _L)
            for k in range(_CHUNK // _L):
                sl = pl.ds(off + i32(k * _L), _L)
                bsl = pl.ds(k * _L, _L)
                tokv = iota + i32(k * _L)
                col = lax.shift_left(idx_all[sl] & i32(7), i32(1))
                w0_v[i32(b), bsl] = plsc.load_gather(wraw_v.at[i32(b)], [tokv, col])
                w1_v[i32(b), bsl] = plsc.load_gather(
                    wraw_v.at[i32(b)], [tokv, col + i32(1)])

            def tok_body(g2, c):
                csplat = jnp.full((_L,), c, i32)
                w0 = plsc.load_gather(w0_v.at[i32(b)], [csplat])
                w1 = plsc.load_gather(w1_v.at[i32(b)], [csplat])
                for d in range(_D // _L):
                    ds = pl.ds(d * _L, _L)
                    o_v[i32(b), c, ds] = w0 * r0_v[i32(b), c, ds] + w1 * r1_v[i32(b), c, ds]
                return c + i32(1)

            lax.fori_loop(0, _CHUNK, tok_body, i32(0))

        issue_gathers(i32(0), 0)

        def pair_body(p, off):
            off = pl.multiple_of(off, 2 * _CHUNK)
            issue_gathers(off + i32(_CHUNK), 1)
            wait_gathers(off, 0)

            @pl.when(off > i32(0))
            def _():
                pltpu.make_async_copy(*out_args(off - i32(2 * _CHUNK), 0)).wait()
            combine(off, 0)
            pltpu.async_copy(*out_args(off, 0))
            @pl.when(off + i32(2 * _CHUNK) < i32(t_per_w))
            def _():
                issue_gathers(off + i32(2 * _CHUNK), 0)
            wait_gathers(off + i32(_CHUNK), 1)

            @pl.when(off > i32(0))
            def _():
                pltpu.make_async_copy(*out_args(off - i32(_CHUNK), 1)).wait()
            combine(off + i32(_CHUNK), 1)
            pltpu.async_copy(*out_args(off + i32(_CHUNK), 1))
            return off + i32(2 * _CHUNK)

        lax.fori_loop(0, n_chunks // 2, pair_body, i32(0))
        pltpu.make_async_copy(*out_args(i32(t_per_w - 2 * _CHUNK), 0)).wait()
        pltpu.make_async_copy(*out_args(i32(t_per_w - _CHUNK), 1)).wait()

    return lookup


def kernel(x, shared_embedding, hash_weights):
    b, s = x.shape
    n_tokens = b * s
    xf = x.reshape(n_tokens).astype(jnp.int32)
    hw16 = hash_weights.reshape(-1, _L)
    lookup = _make_lookup(n_tokens)
    out = lookup(xf, shared_embedding, hw16)
    return out.reshape(b, s, _D)


# speedup vs baseline: 3.4016x; 2.3784x over previous
import functools

import jax
import jax.numpy as jnp
from jax import lax
from jax.experimental import pallas as pl
from jax.experimental.pallas import tpu as pltpu
from jax.experimental.pallas import tpu_sc as plsc

_P = 2147483647
_M = 100000
_A = (48271, 16807)
_B = (12345, 67890)
_D = 64
_NC = 2
_NS = 16
_NW = _NC * _NS
_L = 16


def _hashes(xv):
    i32 = jnp.int32
    xl = xv & i32(0x7FFF)
    xh = lax.shift_right_logical(xv, i32(15))
    out = []
    for a, b in zip(_A, _B):
        u = xh * i32(a)
        t = (lax.shift_right_logical(u, i32(16))
             + lax.shift_left(u & i32(0xFFFF), i32(15))
             + xl * i32(a) + i32(b))
        r = (t & i32(0x7FFFFFFF)) + lax.shift_right_logical(t, i32(31))
        r = jnp.where(r == i32(_P), i32(0), jnp.where(r < i32(0), i32(1), r))
        h = lax.rem(r, i32(_M)) + i32(4)
        out.append(jnp.where(xv < i32(4), xv, h))
    return out


def _make_lookup(seq, batch):
    bpw = batch // _NW
    assert bpw * _NW == batch and bpw % _L == 0 and bpw <= 128
    assert seq % 2 == 0

    mesh = plsc.VectorSubcoreMesh(
        core_axis_name="c", subcore_axis_name="s",
        num_cores=_NC, num_subcores=_NS)

    @functools.partial(
        pl.kernel,
        out_type=jax.ShapeDtypeStruct(
            (seq, _D // 8, batch // 128, 8, 128), jnp.float32),
        mesh=mesh,
        scratch_types=[
            pltpu.VMEM((seq, bpw), jnp.int32),
            pltpu.VMEM((seq, bpw), jnp.int32),
            pltpu.VMEM((seq, bpw), jnp.int32),
            pltpu.VMEM((seq, bpw), jnp.int32),
            pltpu.VMEM((2, bpw, _L), jnp.float32),
            pltpu.VMEM((2, bpw, _L), jnp.float32),
            pltpu.VMEM((2, bpw, _D), jnp.float32),
            pltpu.VMEM((2, bpw, _D), jnp.float32),
            pltpu.VMEM((2, _D // 8, 8, bpw), jnp.float32),
            pltpu.SemaphoreType.DMA,
            pltpu.SemaphoreType.DMA,
            pltpu.SemaphoreType.DMA,
            pltpu.SemaphoreType.DMA,
            pltpu.SemaphoreType.DMA,
            pltpu.SemaphoreType.DMA,
            pltpu.SemaphoreType.DMA,
            pltpu.SemaphoreType.DMA,
            pltpu.SemaphoreType.DMA,
            pltpu.SemaphoreType.DMA,
        ],
        compiler_params=pltpu.CompilerParams(
            needs_layout_passes=False, use_tc_tiling_on_sc=False),
    )
    def lookup(xt_hbm, emb_hbm, w0t_hbm, w1t_hbm, out_hbm,
               idx_s, h0_s, h1_s, g_s, w0raw, w1raw, r0_v, r1_v, o_v,
               sw00, sw01, sw10, sw11, s00, s01, s10, s11, so0, so1):
        i32 = jnp.int32
        wid = lax.axis_index("s") * i32(_NC) + lax.axis_index("c")
        b0 = pl.multiple_of(wid * i32(bpw), bpw)
        sems = ((sw00, sw10, s00, s10, so0), (sw01, sw11, s01, s11, so1))

        pltpu.sync_copy(xt_hbm.at[:, pl.ds(b0, bpw)], idx_s)

        def hash_body(_, r):
            for k in range(bpw // _L):
                sl = pl.ds(k * _L, _L)
                xv = idx_s[r, sl]
                h0, h1 = _hashes(xv)
                h0_s[r, sl] = h0
                h1_s[r, sl] = h1
                g_s[r, sl] = lax.shift_right_logical(xv, i32(4))
            return r + i32(1)

        lax.fori_loop(0, seq, hash_body, i32(0))

        def gather_args(r, p):
            return ((w0t_hbm.at[g_s.at[r]], w0raw.at[i32(p)], sems[p][0]),
                    (w1t_hbm.at[g_s.at[r]], w1raw.at[i32(p)], sems[p][1]),
                    (emb_hbm.at[h0_s.at[r]], r0_v.at[i32(p)], sems[p][2]),
                    (emb_hbm.at[h1_s.at[r]], r1_v.at[i32(p)], sems[p][3]))

        def issue_gathers(r, p):
            for args in gather_args(r, p):
                pltpu.async_copy(*args)

        def wait_gathers(r, p):
            for args in gather_args(r, p):
                pltpu.make_async_copy(*args).wait()

        def out_args(r, p):
            return (o_v.at[i32(p)], out_hbm.at[r, :, wid, :, :],
                    sems[p][4])

        def combine(r, p):
            iota = lax.iota(i32, _L)

            def grp_body(g2, koff):
                koff = pl.multiple_of(koff, _L)
                tokv = iota + koff
                ksl = pl.ds(koff, _L)
                col = idx_s[r, ksl] & i32(15)
                w0 = plsc.load_gather(w0raw.at[i32(p)], [tokv, col])
                w1 = plsc.load_gather(w1raw.at[i32(p)], [tokv, col])
                for d in range(_D):
                    dspl = jnp.full((_L,), d, i32)
                    g0 = plsc.load_gather(r0_v.at[i32(p)], [tokv, dspl])
                    g1 = plsc.load_gather(r1_v.at[i32(p)], [tokv, dspl])
                    o_v[i32(p), d // 8, d % 8, ksl] = w0 * g0 + w1 * g1
                return koff + i32(_L)

            lax.fori_loop(0, bpw // _L, grp_body, i32(0))

        issue_gathers(i32(0), 0)

        def pair_body(q, r):
            issue_gathers(r + i32(1), 1)
            wait_gathers(r, 0)

            @pl.when(r > i32(0))
            def _():
                pltpu.make_async_copy(*out_args(r - i32(2), 0)).wait()
            combine(r, 0)
            pltpu.async_copy(*out_args(r, 0))
            @pl.when(r + i32(2) < i32(seq))
            def _():
                issue_gathers(r + i32(2), 0)
            wait_gathers(r + i32(1), 1)

            @pl.when(r > i32(0))
            def _():
                pltpu.make_async_copy(*out_args(r - i32(1), 1)).wait()
            combine(r + i32(1), 1)
            pltpu.async_copy(*out_args(r + i32(1), 1))
            return r + i32(2)

        lax.fori_loop(0, seq // 2, pair_body, i32(0))
        pltpu.make_async_copy(*out_args(i32(seq - 2), 0)).wait()
        pltpu.make_async_copy(*out_args(i32(seq - 1), 1)).wait()

    return lookup


def kernel(x, shared_embedding, hash_weights):
    b, s = x.shape
    xt = x.T.astype(jnp.int32)
    w0t = hash_weights[:, 0].reshape(-1, _L)
    w1t = hash_weights[:, 1].reshape(-1, _L)
    lookup = _make_lookup(s, b)
    out5 = lookup(xt, shared_embedding, w0t, w1t)
    return jnp.transpose(out5, (2, 4, 0, 1, 3)).reshape(b, s, _D)
